# Initial kernel scaffold; baseline (speedup 1.0000x reference)
import functools

import jax
import jax.numpy as jnp
from jax import lax
from jax.experimental import pallas as pl
from jax.experimental.pallas import tpu as pltpu
from jax.experimental.pallas import tpu_sc as plsc

N = 10000
E = 320000
IN_F = 128
HID = 64
OUT_F = 40
OUT_P = 48

NC, NS = 2, 16
NW = NC * NS
ROWS_PER_S = 640
NP = NS * ROWS_PER_S
CHUNK = 128
NCHUNK = E // CHUNK
CH_PER_W = (NCHUNK + NW - 1) // NW

_MESH = dict(core_axis_name="c", subcore_axis_name="s")


def _worker_ids():
    cid = lax.axis_index("c")
    sid = lax.axis_index("s")
    return cid, sid, sid * NC + cid


@functools.partial(
    pl.kernel,
    out_type=jax.ShapeDtypeStruct((2, NC, NP), jnp.float32),
    mesh=plsc.VectorSubcoreMesh(**_MESH),
    scratch_types=[
        pltpu.VMEM((2, CHUNK), jnp.int32),
        pltpu.VMEM((CHUNK,), jnp.float32),
        pltpu.VMEM((ROWS_PER_S,), jnp.float32),
        pltpu.VMEM_SHARED((NP,), jnp.float32),
        pltpu.VMEM_SHARED((NP,), jnp.float32),
    ],
)
def _deg_kernel(ei_hbm, degs_hbm, idx_v, ones_v, zeros_v, dout_sh, din_sh):
    cid, sid, wid = _worker_ids()

    def fill_ones(i, _):
        ones_v[pl.ds(i * 16, 16)] = jnp.ones((16,), jnp.float32)
        return 0

    lax.fori_loop(0, CHUNK // 16, fill_ones, 0)

    def fill_zeros(i, _):
        zeros_v[pl.ds(i * 16, 16)] = jnp.zeros((16,), jnp.float32)
        return 0

    lax.fori_loop(0, ROWS_PER_S // 16, fill_zeros, 0)

    base = sid * ROWS_PER_S
    pltpu.sync_copy(zeros_v, dout_sh.at[pl.ds(base, ROWS_PER_S)])
    pltpu.sync_copy(zeros_v, din_sh.at[pl.ds(base, ROWS_PER_S)])
    plsc.subcore_barrier()

    def chunk_body(k, _):
        c = wid + k * NW

        @pl.when(c < NCHUNK)
        def _():
            pltpu.sync_copy(ei_hbm.at[:, pl.ds(c * CHUNK, CHUNK)], idx_v)
            pltpu.sync_copy(ones_v, dout_sh.at[idx_v.at[0]], add=True)
            pltpu.sync_copy(ones_v, din_sh.at[idx_v.at[1]], add=True)

        return 0

    lax.fori_loop(0, CH_PER_W, chunk_body, 0)
    plsc.subcore_barrier()
    pltpu.sync_copy(dout_sh.at[pl.ds(base, ROWS_PER_S)],
                    degs_hbm.at[0, cid, pl.ds(base, ROWS_PER_S)])
    pltpu.sync_copy(din_sh.at[pl.ds(base, ROWS_PER_S)],
                    degs_hbm.at[1, cid, pl.ds(base, ROWS_PER_S)])


def _make_agg(F):
    @functools.partial(
        pl.kernel,
        out_type=jax.ShapeDtypeStruct((NC, NP, F), jnp.float32),
        mesh=plsc.VectorSubcoreMesh(**_MESH),
        scratch_types=[
            pltpu.VMEM((2, CHUNK), jnp.int32),
            pltpu.VMEM((CHUNK, F), jnp.float32),
            pltpu.VMEM((CHUNK, F), jnp.float32),
            pltpu.SemaphoreType.DMA,
            pltpu.VMEM_SHARED((NP, F), jnp.float32),
        ],
    )
    def _agg_kernel(h_hbm, ei_hbm, out_hbm, idx_v, msg_v, zrow_v, gsem, agg_sh):
        cid, sid, wid = _worker_ids()

        def fill_zeros(r, _):
            for l in range(F // 16):
                zrow_v[r, pl.ds(l * 16, 16)] = jnp.zeros((16,), jnp.float32)
            return 0

        lax.fori_loop(0, CHUNK, fill_zeros, 0)
        base = sid * ROWS_PER_S
        for t in range(ROWS_PER_S // CHUNK):
            pltpu.sync_copy(zrow_v, agg_sh.at[pl.ds(base + t * CHUNK, CHUNK)])
        plsc.subcore_barrier()

        def chunk_body(k, _):
            c = wid + k * NW

            @pl.when(c < NCHUNK)
            def _():
                pltpu.sync_copy(ei_hbm.at[:, pl.ds(c * CHUNK, CHUNK)], idx_v)
                pltpu.async_copy(h_hbm.at[idx_v.at[0]], msg_v, gsem).wait()
                pltpu.sync_copy(msg_v, agg_sh.at[idx_v.at[1]], add=True)

            return 0

        lax.fori_loop(0, CH_PER_W, chunk_body, 0)
        plsc.subcore_barrier()
        pltpu.sync_copy(agg_sh.at[pl.ds(base, ROWS_PER_S)],
                        out_hbm.at[cid, pl.ds(base, ROWS_PER_S)])

    return _agg_kernel


_agg_hid = _make_agg(HID)
_agg_out = _make_agg(OUT_P)


def _mm1_body(x_ref, w_ref, o_ref):
    o_ref[...] = jnp.dot(x_ref[...], w_ref[...],
                         preferred_element_type=jnp.float32)


def _scale1_body(u_ref, degs_ref, o_ref):
    d = degs_ref[0, 0, :N] + degs_ref[0, 1, :N]
    s = lax.rsqrt(jnp.maximum(d, 1.0))
    o_ref[...] = u_ref[...] * s[:, None]


def _mid_body(a_ref, degs_ref, b1_ref, w2_ref, o_ref):
    agg = a_ref[0, :N, :] + a_ref[1, :N, :]
    din = degs_ref[1, 0, :N] + degs_ref[1, 1, :N]
    dout = degs_ref[0, 0, :N] + degs_ref[0, 1, :N]
    si = lax.rsqrt(jnp.maximum(din, 1.0))
    so = lax.rsqrt(jnp.maximum(dout, 1.0))
    t = jnp.maximum(agg * si[:, None] + b1_ref[0, :][None, :], 0.0)
    o_ref[...] = jnp.dot(t, w2_ref[...],
                         preferred_element_type=jnp.float32) * so[:, None]


def _out_body(a_ref, degs_ref, b2_ref, o_ref):
    agg = a_ref[0, :N, :OUT_F] + a_ref[1, :N, :OUT_F]
    din = degs_ref[1, 0, :N] + degs_ref[1, 1, :N]
    si = lax.rsqrt(jnp.maximum(din, 1.0))
    o_ref[...] = agg * si[:, None] + b2_ref[0, :][None, :]


_mm1 = pl.pallas_call(
    _mm1_body, out_shape=jax.ShapeDtypeStruct((N, HID), jnp.float32))
_scale1 = pl.pallas_call(
    _scale1_body, out_shape=jax.ShapeDtypeStruct((N, HID), jnp.float32))
_mid = pl.pallas_call(
    _mid_body, out_shape=jax.ShapeDtypeStruct((N, OUT_P), jnp.float32))
_out = pl.pallas_call(
    _out_body, out_shape=jax.ShapeDtypeStruct((N, OUT_F), jnp.float32))


def kernel(features, edge_index, W1, b1, W2, b2):
    ei = edge_index.astype(jnp.int32)
    degs = _deg_kernel(ei)
    u = _mm1(features, W1)
    h1s = _scale1(u, degs)
    agg1 = _agg_hid(h1s, ei)
    w2p = jnp.pad(W2, ((0, 0), (0, OUT_P - OUT_F)))
    h2s = _mid(agg1, degs, b1.reshape(1, -1), w2p)
    agg2 = _agg_out(h2s, ei)
    return _out(agg2, degs, b2.reshape(1, -1))


# speedup vs baseline: 8.9578x; 8.9578x over previous
import functools

import jax
import jax.numpy as jnp
from jax import lax
from jax.experimental import pallas as pl
from jax.experimental.pallas import tpu as pltpu
from jax.experimental.pallas import tpu_sc as plsc

N = 10000
E = 320000
IN_F = 128
HID = 64
OUT_F = 40
OUT_P = 48

NC, NS = 2, 16
NW = NC * NS
ROWS_PER_S = 640
NP = NS * ROWS_PER_S
CHUNK = 128
NCHUNK = E // CHUNK
CH_PER_W = (NCHUNK + NW - 1) // NW

_MESH = dict(core_axis_name="c", subcore_axis_name="s")
_SC_PARAMS = pltpu.CompilerParams(use_tc_tiling_on_sc=False)


def _worker_ids():
    cid = lax.axis_index("c")
    sid = lax.axis_index("s")
    return cid, sid, sid * NC + cid


@functools.partial(
    pl.kernel,
    out_type=jax.ShapeDtypeStruct((2, NC, NP), jnp.float32),
    mesh=plsc.VectorSubcoreMesh(**_MESH),
    compiler_params=_SC_PARAMS,
    scratch_types=[
        pltpu.VMEM((2, CHUNK), jnp.int32),
        pltpu.VMEM((CHUNK,), jnp.float32),
        pltpu.VMEM((ROWS_PER_S,), jnp.float32),
        pltpu.VMEM_SHARED((NP,), jnp.float32),
        pltpu.VMEM_SHARED((NP,), jnp.float32),
    ],
)
def _deg_kernel(ei_hbm, degs_hbm, idx_v, ones_v, zeros_v, dout_sh, din_sh):
    cid, sid, wid = _worker_ids()

    def fill_ones(i, _):
        ones_v[pl.ds(i * 16, 16)] = jnp.ones((16,), jnp.float32)
        return 0

    lax.fori_loop(0, CHUNK // 16, fill_ones, 0)

    def fill_zeros(i, _):
        zeros_v[pl.ds(i * 16, 16)] = jnp.zeros((16,), jnp.float32)
        return 0

    lax.fori_loop(0, ROWS_PER_S // 16, fill_zeros, 0)

    base = sid * ROWS_PER_S
    pltpu.sync_copy(zeros_v, dout_sh.at[pl.ds(base, ROWS_PER_S)])
    pltpu.sync_copy(zeros_v, din_sh.at[pl.ds(base, ROWS_PER_S)])
    plsc.subcore_barrier()

    def chunk_body(k, _):
        c = wid + k * NW

        @pl.when(c < NCHUNK)
        def _():
            pltpu.sync_copy(ei_hbm.at[:, pl.ds(c * CHUNK, CHUNK)], idx_v)
            pltpu.sync_copy(ones_v, dout_sh.at[idx_v.at[0]], add=True)
            pltpu.sync_copy(ones_v, din_sh.at[idx_v.at[1]], add=True)

        return 0

    lax.fori_loop(0, CH_PER_W, chunk_body, 0)
    plsc.subcore_barrier()
    pltpu.sync_copy(dout_sh.at[pl.ds(base, ROWS_PER_S)],
                    degs_hbm.at[0, cid, pl.ds(base, ROWS_PER_S)])
    pltpu.sync_copy(din_sh.at[pl.ds(base, ROWS_PER_S)],
                    degs_hbm.at[1, cid, pl.ds(base, ROWS_PER_S)])


def _make_agg(F):
    @functools.partial(
        pl.kernel,
        out_type=jax.ShapeDtypeStruct((NC, NP, F), jnp.float32),
        mesh=plsc.VectorSubcoreMesh(**_MESH),
        compiler_params=_SC_PARAMS,
        scratch_types=[
            pltpu.VMEM((2, CHUNK), jnp.int32),
            pltpu.VMEM((CHUNK, F), jnp.float32),
            pltpu.VMEM((CHUNK, F), jnp.float32),
            pltpu.SemaphoreType.DMA,
            pltpu.VMEM_SHARED((NP, F), jnp.float32),
        ],
    )
    def _agg_kernel(h_hbm, ei_hbm, out_hbm, idx_v, msg_v, zrow_v, gsem, agg_sh):
        cid, sid, wid = _worker_ids()

        def fill_zeros(r, _):
            for l in range(F // 16):
                zrow_v[r, pl.ds(l * 16, 16)] = jnp.zeros((16,), jnp.float32)
            return 0

        lax.fori_loop(0, CHUNK, fill_zeros, 0)
        base = sid * ROWS_PER_S
        for t in range(ROWS_PER_S // CHUNK):
            pltpu.sync_copy(zrow_v, agg_sh.at[pl.ds(base + t * CHUNK, CHUNK)])
        plsc.subcore_barrier()

        def chunk_body(k, _):
            c = wid + k * NW

            @pl.when(c < NCHUNK)
            def _():
                pltpu.sync_copy(ei_hbm.at[:, pl.ds(c * CHUNK, CHUNK)], idx_v)
                pltpu.async_copy(h_hbm.at[idx_v.at[0]], msg_v, gsem).wait()
                pltpu.sync_copy(msg_v, agg_sh.at[idx_v.at[1]], add=True)

            return 0

        lax.fori_loop(0, CH_PER_W, chunk_body, 0)
        plsc.subcore_barrier()
        pltpu.sync_copy(agg_sh.at[pl.ds(base, ROWS_PER_S)],
                        out_hbm.at[cid, pl.ds(base, ROWS_PER_S)])

    return _agg_kernel


_agg_hid = _make_agg(HID)
_agg_out = _make_agg(OUT_P)


def _mm1_body(x_ref, w_ref, o_ref):
    o_ref[...] = jnp.dot(x_ref[...], w_ref[...],
                         preferred_element_type=jnp.float32)


def _scale1_body(u_ref, degs_ref, o_ref):
    d = degs_ref[0, 0, :N] + degs_ref[0, 1, :N]
    s = lax.rsqrt(jnp.maximum(d, 1.0))
    o_ref[...] = u_ref[...] * s[:, None]


def _mid_body(a_ref, degs_ref, b1_ref, w2_ref, o_ref):
    agg = a_ref[0, :N, :] + a_ref[1, :N, :]
    din = degs_ref[1, 0, :N] + degs_ref[1, 1, :N]
    dout = degs_ref[0, 0, :N] + degs_ref[0, 1, :N]
    si = lax.rsqrt(jnp.maximum(din, 1.0))
    so = lax.rsqrt(jnp.maximum(dout, 1.0))
    t = jnp.maximum(agg * si[:, None] + b1_ref[0, :][None, :], 0.0)
    o_ref[...] = jnp.dot(t, w2_ref[...],
                         preferred_element_type=jnp.float32) * so[:, None]


def _out_body(a_ref, degs_ref, b2_ref, o_ref):
    agg = a_ref[0, :N, :OUT_F] + a_ref[1, :N, :OUT_F]
    din = degs_ref[1, 0, :N] + degs_ref[1, 1, :N]
    si = lax.rsqrt(jnp.maximum(din, 1.0))
    o_ref[...] = agg * si[:, None] + b2_ref[0, :][None, :]


_mm1 = pl.pallas_call(
    _mm1_body, out_shape=jax.ShapeDtypeStruct((N, HID), jnp.float32))
_scale1 = pl.pallas_call(
    _scale1_body, out_shape=jax.ShapeDtypeStruct((N, HID), jnp.float32))
_mid = pl.pallas_call(
    _mid_body, out_shape=jax.ShapeDtypeStruct((N, OUT_P), jnp.float32))
_out = pl.pallas_call(
    _out_body, out_shape=jax.ShapeDtypeStruct((N, OUT_F), jnp.float32))


def kernel(features, edge_index, W1, b1, W2, b2):
    ei = edge_index.astype(jnp.int32)
    degs = _deg_kernel(ei)
    u = _mm1(features, W1)
    h1s = _scale1(u, degs)
    agg1 = _agg_hid(h1s, ei)
    w2p = jnp.pad(W2, ((0, 0), (0, OUT_P - OUT_F)))
    h2s = _mid(agg1, degs, b1.reshape(1, -1), w2p)
    agg2 = _agg_out(h2s, ei)
    return _out(agg2, degs, b2.reshape(1, -1))
